# baseline (speedup 1.0000x reference)
import functools

import jax
import jax.numpy as jnp
import numpy as np
from jax.experimental import pallas as pl
from jax.experimental.pallas import tpu as pltpu

EPS = 1e-7

_N, _M, _D, _K, _H, _W = 16, 1, 8, 1024, 32, 32
_P = _N * _H * _W
_TP = 256
_GRID = _P // _TP

_R1 = (13, 15, 26, 6)
_R2 = (17, 29, 16, 24)


def _tf_round(x0, x1, r):
    x0 = x0 + x1
    x1 = (x1 << np.uint32(r)) | (x1 >> np.uint32(32 - r))
    x1 = x0 ^ x1
    return x0, x1


def _threefry_bits(k1, k2, counts):
    ks0, ks1 = k1, k2
    ks2 = ks0 ^ ks1 ^ np.uint32(0x1BD11BDA)
    x0 = ks0
    x1 = counts + ks1
    for r in _R1:
        x0, x1 = _tf_round(x0, x1, r)
    x0 = x0 + ks1
    x1 = x1 + (ks2 + np.uint32(1))
    for r in _R2:
        x0, x1 = _tf_round(x0, x1, r)
    x0 = x0 + ks2
    x1 = x1 + (ks0 + np.uint32(2))
    for r in _R1:
        x0, x1 = _tf_round(x0, x1, r)
    x0 = x0 + ks0
    x1 = x1 + (ks1 + np.uint32(3))
    for r in _R2:
        x0, x1 = _tf_round(x0, x1, r)
    x0 = x0 + ks1
    x1 = x1 + (ks2 + np.uint32(4))
    for r in _R1:
        x0, x1 = _tf_round(x0, x1, r)
    x0 = x0 + ks2
    x1 = x1 + (ks0 + np.uint32(5))
    return x0 ^ x1


def _bits_to_unit_float(bits):
    fb = (bits >> np.uint32(9)) | np.uint32(0x3F800000)
    return jax.lax.bitcast_convert_type(fb, jnp.float32) - jnp.float32(1.0)


def _quant_kernel(key_ref, t_ref, x_ref, cbt_ref, cb_ref, freq_ref,
                  out_ref, code_ref):
    i = pl.program_id(0)
    bits_log2 = jnp.float32(np.log2(_K))
    scale = jnp.float32(np.sqrt(_K))

    xt = x_ref[...]
    cbt = cbt_ref[...]
    x2 = jnp.sum(xt * xt, axis=1, keepdims=True)
    c2 = jnp.sum(cbt * cbt, axis=0, keepdims=True)
    inter = jnp.dot(xt, cbt, preferred_element_type=jnp.float32)
    dist = (x2 + c2) - jnp.float32(2.0) * inter
    t = jnp.maximum(t_ref[0, 0], jnp.float32(EPS))
    logit = ((-dist) / scale) * t

    freq = freq_ref[...]
    code_usage = jnp.clip(jnp.mean((freq > jnp.float32(EPS)).astype(jnp.float32)),
                          jnp.float32(0.0), jnp.float32(1.0))
    expo = -(bits_log2 - jnp.float32(1.0)) * code_usage * code_usage + bits_log2

    base = (i * np.int32(_TP * _K)).astype(jnp.int32)
    row = jax.lax.broadcasted_iota(jnp.int32, (_TP, _K), 0)
    col = jax.lax.broadcasted_iota(jnp.int32, (_TP, _K), 1)
    counts = (base + row * np.int32(_K) + col).astype(jnp.uint32)

    rbits = _threefry_bits(key_ref[0, 0], key_ref[0, 1], counts)
    rflt = _bits_to_unit_float(rbits)
    r = jnp.maximum(jnp.float32(0.0),
                    rflt * jnp.float32(1.0) + jnp.float32(0.0))
    random_mask = (r ** expo) < freq
    logit = jnp.where(random_mask, logit - jnp.float32(1e9), logit)

    lmax = jnp.max(logit, axis=1, keepdims=True)
    code = jnp.min(jnp.where(logit == lmax, col, np.int32(_K)), axis=1)
    code_ref[...] = code.reshape(1, 1, _TP)

    ubits = _threefry_bits(key_ref[1, 0], key_ref[1, 1], counts)
    uflt = _bits_to_unit_float(ubits)
    mn = jnp.float32(1e-20)
    u = jnp.maximum(mn, uflt * (jnp.float32(1.0) - mn) + mn)
    gumbels = -jnp.log(-jnp.log(u))
    z = (logit + gumbels) / jnp.float32(1.0)
    zmax = jnp.max(z, axis=1, keepdims=True)
    unnorm = jnp.exp(z - zmax)
    ssum = jnp.sum(unnorm, axis=1, keepdims=True)
    ysoft = unnorm / ssum
    ymax = jnp.max(ysoft, axis=1, keepdims=True)
    idx = jnp.min(jnp.where(ysoft == ymax, col, np.int32(_K)),
                  axis=1, keepdims=True)
    yhard = (col == idx).astype(jnp.float32)
    sample = (yhard - ysoft) + ysoft

    out_ref[...] = jnp.dot(sample, cb_ref[...],
                           preferred_element_type=jnp.float32)


@jax.jit
def kernel(x, codebook, temperature, freqEMA):
    n, c, h, w = x.shape
    m, k, d = codebook.shape
    p = n * h * w

    x2d = x.reshape(n, m * d, h * w).transpose(0, 2, 1).reshape(p, m * d)
    cb2d = codebook.reshape(k, d)
    cbt = cb2d.T
    freq2d = freqEMA.reshape(1, k)
    t2d = temperature.reshape(1, 1)

    ku, kg = jax.random.split(jax.random.key(42))
    keys = jnp.stack([jax.random.key_data(ku),
                      jax.random.key_data(kg)]).astype(jnp.uint32)

    out2d, code3d = pl.pallas_call(
        _quant_kernel,
        grid=(_GRID,),
        in_specs=[
            pl.BlockSpec(memory_space=pltpu.MemorySpace.SMEM),
            pl.BlockSpec(memory_space=pltpu.MemorySpace.SMEM),
            pl.BlockSpec((_TP, d), lambda i: (i, 0)),
            pl.BlockSpec((d, k), lambda i: (0, 0)),
            pl.BlockSpec((k, d), lambda i: (0, 0)),
            pl.BlockSpec((1, k), lambda i: (0, 0)),
        ],
        out_specs=[
            pl.BlockSpec((_TP, d), lambda i: (i, 0)),
            pl.BlockSpec((1, 1, _TP), lambda i: (i, 0, 0)),
        ],
        out_shape=[
            jax.ShapeDtypeStruct((p, d), jnp.float32),
            jax.ShapeDtypeStruct((_GRID, 1, _TP), jnp.int32),
        ],
        compiler_params=pltpu.CompilerParams(
            dimension_semantics=("parallel",)),
    )(keys, t2d, x2d, cbt, cb2d, freq2d)

    out = out2d.reshape(n, h, w, m * d).transpose(0, 3, 1, 2)
    code = code3d.reshape(n, m, h, w)
    return out, code
